# baseline (speedup 1.0000x reference)
import functools

import jax
import jax.numpy as jnp
from jax import lax
from jax.experimental import pallas as pl
from jax.experimental.pallas import tpu as pltpu
from jax.experimental.pallas import tpu_sc as plsc

_N = 50000
_NBX = 256
_NBY = 256
_BS = 2.0
_TGT = 0.9 * (_BS * _BS)
_NIMP = 7
_NC = 2
_NS = 16
_L = 16
_NW = _NC * _NS
_C = 1568
_NV = _C // _L
_ROWS = _NBX + 1
_PRIV = _ROWS * _NBY


def _sc_body(pos, sx, sy, axr, bxr, cxr, ayr, byr, cyr, out_hbm,
             s_px, s_py, s_sx, s_sy, s_ax, s_bx, s_cx, s_ay, s_by, s_cy,
             priv, shared, idx_a, idx_b, sem):
  cid = lax.axis_index("c")
  sid = lax.axis_index("s")
  wid = sid * _NC + cid
  lo = wid * _C
  base = jnp.minimum(lo, _N - _C)

  cps = [
      pltpu.async_copy(pos.at[pl.ds(base, _C)], s_px, sem),
      pltpu.async_copy(pos.at[pl.ds(base + _N, _C)], s_py, sem),
      pltpu.async_copy(sx.at[pl.ds(base, _C)], s_sx, sem),
      pltpu.async_copy(sy.at[pl.ds(base, _C)], s_sy, sem),
      pltpu.async_copy(axr.at[pl.ds(base, _C)], s_ax, sem),
      pltpu.async_copy(bxr.at[pl.ds(base, _C)], s_bx, sem),
      pltpu.async_copy(cxr.at[pl.ds(base, _C)], s_cx, sem),
      pltpu.async_copy(ayr.at[pl.ds(base, _C)], s_ay, sem),
      pltpu.async_copy(byr.at[pl.ds(base, _C)], s_by, sem),
      pltpu.async_copy(cyr.at[pl.ds(base, _C)], s_cy, sem),
  ]

  for j in range(128 // _L):
    iota = lax.iota(jnp.int32, _L)
    idx_a[pl.ds(j * _L, _L)] = iota + (j * _L)
    idx_b[pl.ds(j * _L, _L)] = iota + (j * _L + 128)

  zero = jnp.zeros((_L,), jnp.float32)

  def _zrow(i, carry):
    for j in range(_NBY // _L):
      priv[i, pl.ds(j * _L, _L)] = zero
    return carry

  lax.fori_loop(0, _NBX, _zrow, None)

  @pl.when(sid == 0)
  def _():
    pltpu.sync_copy(priv.at[pl.ds(0, _NBX)], shared)

  for cp in cps:
    cp.wait()

  zero = jnp.zeros((_L,), jnp.float32)

  @pl.when(lo != base)
  def _():
    for j in range((_NW * _C - _N) // _L):
      s_cx[pl.ds(j * _L, _L)] = zero

  zrow = lax.iota(jnp.int32, _L) * 0

  def _node_vreg(i, carry):
    off = i * _L
    sl = pl.ds(off, _L)
    halfx = s_sx[sl] * 0.5
    xc = s_px[sl] + halfx
    r1x = halfx + _BS
    r2x = halfx + 2.0 * _BS
    w = s_cx[sl] * s_cy[sl]
    bwx = s_bx[sl] * w
    abwx = (s_ax[sl] + s_bx[sl]) * w
    tx = (xc - r2x) * (1.0 / _BS)
    blx = jnp.maximum(tx.astype(jnp.int32), 0)
    t0x = xc - (2 * blx + 1).astype(jnp.float32)
    blx8 = blx << 8

    halfy = s_sy[sl] * 0.5
    yc = s_py[sl] + halfy
    r1y = halfy + _BS
    r2y = halfy + 2.0 * _BS
    aby = s_ay[sl] + s_by[sl]
    byv = s_by[sl]
    ty = (yc - r2y) * (1.0 / _BS)
    bly = jnp.maximum(ty.astype(jnp.int32), 0)
    t0y = yc - (2 * bly + 1).astype(jnp.float32)

    wpx = []
    ix8 = []
    for k in range(_NIMP):
      d = jnp.abs(t0x - (2.0 * k))
      m2 = jnp.minimum(d, r2x) - r2x
      m1 = jnp.minimum(d, r1x) - r1x
      wpx.append(bwx * (m2 * m2) - abwx * (m1 * m1))
      ix8.append(jnp.minimum(blx8 + (k << 8), _NBX << 8))
    py = []
    iyc = []
    for k in range(_NIMP):
      idx = bly + k
      d = jnp.abs(t0y - (2.0 * k))
      d = jnp.where(idx < _NBY, d, 1e9)
      m2 = jnp.minimum(d, r2y) - r2y
      m1 = jnp.minimum(d, r1y) - r1y
      py.append(byv * (m2 * m2) - aby * (m1 * m1))
      iyc.append(jnp.minimum(idx, _NBY - 1))
    for k1 in range(_NIMP):
      for k2 in range(_NIMP):
        plsc.addupdate_scatter(priv, [zrow, ix8[k1] + iyc[k2]],
                               wpx[k1] * py[k2])
    return carry

  lax.fori_loop(0, _NV, _node_vreg, None)

  plsc.subcore_barrier()
  m1 = pltpu.async_copy(priv.at[pl.ds(0, 128)], shared.at[idx_a],
                        sem, add=True)
  m2 = pltpu.async_copy(priv.at[pl.ds(128, 128)],
                        shared.at[idx_b], sem, add=True)
  m1.wait()
  m2.wait()
  plsc.subcore_barrier()

  for j in range(2):
    ts = sid * 2 + j
    for c in range(2):
      pltpu.sync_copy(shared.at[pl.ds(ts * 8, 8), pl.ds(c * 128, 128)],
                      out_hbm.at[cid, ts, c])


_sc_scatter = functools.partial(
    pl.kernel,
    out_type=jax.ShapeDtypeStruct((_NC, _NBX // 8, 2, 8, 128), jnp.float32),
    mesh=plsc.VectorSubcoreMesh(core_axis_name="c", subcore_axis_name="s"),
    compiler_params=pltpu.CompilerParams(use_tc_tiling_on_sc=False,
                                         needs_layout_passes=False),
    scratch_types=(
        [pltpu.VMEM((_C,), jnp.float32) for _ in range(10)]
        + [
            pltpu.VMEM((_ROWS, _NBY), jnp.float32),
            pltpu.VMEM_SHARED((_NBX, _NBY), jnp.float32),
            pltpu.VMEM((128,), jnp.int32),
            pltpu.VMEM((128,), jnp.int32),
            pltpu.SemaphoreType.DMA,
        ]
    ),
)(_sc_body)


def _cost_body(m_ref, init_ref, out_ref):
  def blk(t, acc):
    r0 = t * 8
    a0 = (m_ref[0, t, 0] + m_ref[1, t, 0]
          + init_ref[pl.ds(r0, 8), pl.ds(0, 128)] - _TGT)
    a1 = (m_ref[0, t, 1] + m_ref[1, t, 1]
          + init_ref[pl.ds(r0, 8), pl.ds(128, 128)] - _TGT)
    return acc + a0 * a0 + a1 * a1

  acc = lax.fori_loop(0, _NBX // 8, blk, jnp.zeros((8, 128), jnp.float32))
  out_ref[...] = jnp.sum(acc, keepdims=True)


_tc_cost = pl.pallas_call(
    _cost_body,
    out_shape=jax.ShapeDtypeStruct((1, 1), jnp.float32),
)


def kernel(pos, node_size_x, node_size_y, ax, bx, cx, ay, by, cy,
           initial_density_map):
  maps = _sc_scatter(pos, node_size_x, node_size_y, ax, bx, cx, ay, by, cy)
  cost = _tc_cost(maps, initial_density_map)
  return cost[0, 0]


# speedup vs baseline: 1.0424x; 1.0424x over previous
import functools

import jax
import jax.numpy as jnp
from jax import lax
from jax.experimental import pallas as pl
from jax.experimental.pallas import tpu as pltpu
from jax.experimental.pallas import tpu_sc as plsc

_N = 50000
_NBX = 256
_NBY = 256
_BS = 2.0
_TGT = 0.9 * (_BS * _BS)
_NIMP = 7
_NC = 2
_NS = 16
_L = 16
_NW = _NC * _NS
_C = 1568
_NV = _C // _L
_ROWS = _NBX + 1
_PRIV = _ROWS * _NBY


def _sc_body(pos, sx, sy, axr, bxr, cxr, ayr, byr, cyr, out_hbm,
             s_px, s_py, s_sx, s_sy, s_ax, s_bx, s_cx, s_ay, s_by, s_cy,
             priv, shared, idx_a, idx_b, sem):
  cid = lax.axis_index("c")
  sid = lax.axis_index("s")
  wid = sid * _NC + cid
  lo = wid * _C
  base = jnp.minimum(lo, _N - _C)

  cps = [
      pltpu.async_copy(pos.at[pl.ds(base, _C)], s_px, sem),
      pltpu.async_copy(pos.at[pl.ds(base + _N, _C)], s_py, sem),
      pltpu.async_copy(sx.at[pl.ds(base, _C)], s_sx, sem),
      pltpu.async_copy(sy.at[pl.ds(base, _C)], s_sy, sem),
      pltpu.async_copy(axr.at[pl.ds(base, _C)], s_ax, sem),
      pltpu.async_copy(bxr.at[pl.ds(base, _C)], s_bx, sem),
      pltpu.async_copy(cxr.at[pl.ds(base, _C)], s_cx, sem),
      pltpu.async_copy(ayr.at[pl.ds(base, _C)], s_ay, sem),
      pltpu.async_copy(byr.at[pl.ds(base, _C)], s_by, sem),
      pltpu.async_copy(cyr.at[pl.ds(base, _C)], s_cy, sem),
  ]

  for j in range(128 // _L):
    iota = lax.iota(jnp.int32, _L)
    idx_a[pl.ds(j * _L, _L)] = iota + (j * _L)
    idx_b[pl.ds(j * _L, _L)] = iota + (j * _L + 128)

  zero = jnp.zeros((_L,), jnp.float32)

  def _zrow(i, carry):
    for j in range(_NBY // _L):
      priv[i, pl.ds(j * _L, _L)] = zero
    return carry

  lax.fori_loop(0, _NBX, _zrow, None)

  @pl.when(sid == 0)
  def _():
    pltpu.sync_copy(priv.at[pl.ds(0, _NBX)], shared)

  for cp in cps:
    cp.wait()

  zero = jnp.zeros((_L,), jnp.float32)

  @pl.when(lo != base)
  def _():
    for j in range((_NW * _C - _N) // _L):
      s_cx[pl.ds(j * _L, _L)] = zero

  zrow = lax.iota(jnp.int32, _L) * 0

  def _node_vreg(i, carry):
    off = i * _L
    sl = pl.ds(off, _L)
    halfx = s_sx[sl] * 0.5
    xc = s_px[sl] + halfx
    r1x = halfx + _BS
    r2x = halfx + 2.0 * _BS
    w = s_cx[sl] * s_cy[sl]
    bwx = s_bx[sl] * w
    abwx = (s_ax[sl] + s_bx[sl]) * w
    tx = (xc - r2x) * (1.0 / _BS)
    blx = jnp.maximum(tx.astype(jnp.int32), 0)
    t0x = xc - (2 * blx + 1).astype(jnp.float32)
    blx8 = blx << 8

    halfy = s_sy[sl] * 0.5
    yc = s_py[sl] + halfy
    r1y = halfy + _BS
    r2y = halfy + 2.0 * _BS
    aby = s_ay[sl] + s_by[sl]
    byv = s_by[sl]
    ty = (yc - r2y) * (1.0 / _BS)
    bly = jnp.maximum(ty.astype(jnp.int32), 0)
    t0y = yc - (2 * bly + 1).astype(jnp.float32)

    wpx = []
    ix8 = []
    for k in range(_NIMP):
      d = jnp.abs(t0x - (2.0 * k))
      m2 = jnp.minimum(d, r2x) - r2x
      m1 = jnp.minimum(d, r1x) - r1x
      wpx.append(bwx * (m2 * m2) - abwx * (m1 * m1))
      ix8.append(jnp.minimum(blx8 + (k << 8), _NBX << 8))
    py = []
    iyc = []
    for k in range(_NIMP):
      idx = bly + k
      d = jnp.abs(t0y - (2.0 * k))
      d = jnp.where(idx < _NBY, d, 1e9)
      m2 = jnp.minimum(d, r2y) - r2y
      m1 = jnp.minimum(d, r1y) - r1y
      py.append(byv * (m2 * m2) - aby * (m1 * m1))
      iyc.append(jnp.minimum(idx, _NBY - 1))
    for k1 in range(_NIMP):
      for k2 in range(_NIMP):
        plsc.addupdate_scatter(priv, [zrow, ix8[k1] + iyc[k2]],
                               wpx[k1] * py[k2])
    return carry

  lax.fori_loop(0, _NV, _node_vreg, None)

  plsc.subcore_barrier()
  m1 = pltpu.async_copy(priv.at[pl.ds(0, 128)], shared.at[idx_a],
                        sem, add=True)
  m2 = pltpu.async_copy(priv.at[pl.ds(128, 128)],
                        shared.at[idx_b], sem, add=True)
  m1.wait()
  m2.wait()
  plsc.subcore_barrier()

  ocp = []
  for j in range(2):
    ts = sid * 2 + j
    for c in range(2):
      ocp.append(pltpu.async_copy(
          shared.at[pl.ds(ts * 8, 8), pl.ds(c * 128, 128)],
          out_hbm.at[cid, ts, c], sem))
  for cp in ocp:
    cp.wait()


_sc_scatter = functools.partial(
    pl.kernel,
    out_type=jax.ShapeDtypeStruct((_NC, _NBX // 8, 2, 8, 128), jnp.float32),
    mesh=plsc.VectorSubcoreMesh(core_axis_name="c", subcore_axis_name="s"),
    compiler_params=pltpu.CompilerParams(use_tc_tiling_on_sc=False,
                                         needs_layout_passes=False),
    scratch_types=(
        [pltpu.VMEM((_C,), jnp.float32) for _ in range(10)]
        + [
            pltpu.VMEM((_ROWS, _NBY), jnp.float32),
            pltpu.VMEM_SHARED((_NBX, _NBY), jnp.float32),
            pltpu.VMEM((128,), jnp.int32),
            pltpu.VMEM((128,), jnp.int32),
            pltpu.SemaphoreType.DMA,
        ]
    ),
)(_sc_body)


def _cost_body(m_ref, init_ref, out_ref):
  def blk(t, acc):
    r0 = t * 8
    a0 = (m_ref[0, t, 0] + m_ref[1, t, 0]
          + init_ref[pl.ds(r0, 8), pl.ds(0, 128)] - _TGT)
    a1 = (m_ref[0, t, 1] + m_ref[1, t, 1]
          + init_ref[pl.ds(r0, 8), pl.ds(128, 128)] - _TGT)
    return acc + a0 * a0 + a1 * a1

  acc = lax.fori_loop(0, _NBX // 8, blk, jnp.zeros((8, 128), jnp.float32))
  out_ref[...] = jnp.sum(acc, keepdims=True)


_tc_cost = pl.pallas_call(
    _cost_body,
    out_shape=jax.ShapeDtypeStruct((1, 1), jnp.float32),
)


def kernel(pos, node_size_x, node_size_y, ax, bx, cx, ay, by, cy,
           initial_density_map):
  maps = _sc_scatter(pos, node_size_x, node_size_y, ax, bx, cx, ay, by, cy)
  cost = _tc_cost(maps, initial_density_map)
  return cost[0, 0]
